# initial kernel scaffold (unmeasured)
import jax
import jax.numpy as jnp
from jax import lax
from jax.experimental import pallas as pl
from jax.experimental.pallas import tpu as pltpu

N_DEV = 4
N_PER = 2048
D = 1024
E = 32
E_LOC = 8
CAP = 204
SLOTS = E_LOC * CAP
R_ROWS = N_PER // 128


def _ring_ag(x, r2):

    def body(x_ref, r_ref, xall_ref, rall_ref, sx, rx, sr, rr):
        my = lax.axis_index("i")
        left = lax.rem(my + N_DEV - 1, N_DEV)
        right = lax.rem(my + 1, N_DEV)

        barrier_sem = pltpu.get_barrier_semaphore()
        for nbr in (left, right):
            pl.semaphore_signal(
                barrier_sem, inc=1, device_id=(nbr,),
                device_id_type=pl.DeviceIdType.MESH,
            )
        pl.semaphore_wait(barrier_sem, 2)

        xall_ref[pl.ds(my * N_PER, N_PER), :] = x_ref[:, :].astype(jnp.bfloat16)
        rall_ref[pl.ds(my * R_ROWS, R_ROWS), :] = r_ref[:, :]

        for h in range(N_DEV - 1):
            sb = lax.rem(my + N_DEV - h, N_DEV)
            rdma_x = pltpu.make_async_remote_copy(
                src_ref=xall_ref.at[pl.ds(sb * N_PER, N_PER), :],
                dst_ref=xall_ref.at[pl.ds(sb * N_PER, N_PER), :],
                send_sem=sx.at[h],
                recv_sem=rx.at[h],
                device_id=(right,),
                device_id_type=pl.DeviceIdType.MESH,
            )
            rdma_r = pltpu.make_async_remote_copy(
                src_ref=rall_ref.at[pl.ds(sb * R_ROWS, R_ROWS), :],
                dst_ref=rall_ref.at[pl.ds(sb * R_ROWS, R_ROWS), :],
                send_sem=sr.at[h],
                recv_sem=rr.at[h],
                device_id=(right,),
                device_id_type=pl.DeviceIdType.MESH,
            )
            rdma_x.start()
            rdma_r.start()
            rdma_x.wait()
            rdma_r.wait()

    return pl.pallas_call(
        body,
        out_shape=(
            jax.ShapeDtypeStruct((N_DEV * N_PER, D), jnp.bfloat16),
            jax.ShapeDtypeStruct((N_DEV * R_ROWS, 128), jnp.int32),
        ),
        in_specs=[
            pl.BlockSpec(memory_space=pltpu.VMEM),
            pl.BlockSpec(memory_space=pltpu.VMEM),
        ],
        out_specs=(
            pl.BlockSpec(memory_space=pltpu.VMEM),
            pl.BlockSpec(memory_space=pltpu.VMEM),
        ),
        scratch_shapes=[
            pltpu.SemaphoreType.DMA((N_DEV - 1,)),
            pltpu.SemaphoreType.DMA((N_DEV - 1,)),
            pltpu.SemaphoreType.DMA((N_DEV - 1,)),
            pltpu.SemaphoreType.DMA((N_DEV - 1,)),
        ],
        compiler_params=pltpu.CompilerParams(collective_id=0),
    )(x, r2)


def _expert_mm_ag(xg, expert_W):

    def body(xg_ref, ew_ref, yall_ref, wbuf, csem, ss, rs):
        my = lax.axis_index("i")
        left = lax.rem(my + N_DEV - 1, N_DEV)
        right = lax.rem(my + 1, N_DEV)

        barrier_sem = pltpu.get_barrier_semaphore()
        for nbr in (left, right):
            pl.semaphore_signal(
                barrier_sem, inc=1, device_id=(nbr,),
                device_id_type=pl.DeviceIdType.MESH,
            )
        pl.semaphore_wait(barrier_sem, 2)

        for le in range(E_LOC):
            cp = pltpu.make_async_copy(ew_ref.at[le], wbuf, csem)
            cp.start()
            cp.wait()
            w16 = wbuf[:, :].astype(jnp.bfloat16)
            y = jnp.dot(
                xg_ref[pl.ds(le * CAP, CAP), :], w16,
                preferred_element_type=jnp.float32,
            )
            yall_ref[pl.ds(my * SLOTS + le * CAP, CAP), :] = y.astype(jnp.bfloat16)

        for h in range(N_DEV - 1):
            sb = lax.rem(my + N_DEV - h, N_DEV)
            rdma = pltpu.make_async_remote_copy(
                src_ref=yall_ref.at[pl.ds(sb * SLOTS, SLOTS), :],
                dst_ref=yall_ref.at[pl.ds(sb * SLOTS, SLOTS), :],
                send_sem=ss.at[h],
                recv_sem=rs.at[h],
                device_id=(right,),
                device_id_type=pl.DeviceIdType.MESH,
            )
            rdma.start()
            rdma.wait()

    return pl.pallas_call(
        body,
        out_shape=jax.ShapeDtypeStruct((N_DEV * SLOTS, D), jnp.bfloat16),
        in_specs=[
            pl.BlockSpec(memory_space=pltpu.VMEM),
            pl.BlockSpec(memory_space=pltpu.ANY),
        ],
        out_specs=pl.BlockSpec(memory_space=pltpu.VMEM),
        scratch_shapes=[
            pltpu.VMEM((D, D), jnp.float32),
            pltpu.SemaphoreType.DMA,
            pltpu.SemaphoreType.DMA((N_DEV - 1,)),
            pltpu.SemaphoreType.DMA((N_DEV - 1,)),
        ],
        compiler_params=pltpu.CompilerParams(collective_id=1),
    )(xg, expert_W)


def kernel(x, router_W, route_idx, expert_W):
    del router_W
    my = lax.axis_index("i")

    x_all, rall = _ring_ag(x, route_idx.reshape(R_ROWS, 128))
    r_all = rall.reshape(N_DEV * N_PER)

    oh = (r_all[:, None] == jnp.arange(E, dtype=jnp.int32)[None, :]).astype(jnp.int32)
    rank = jnp.take_along_axis(jnp.cumsum(oh, axis=0), r_all[:, None], axis=1)[:, 0] - 1
    kept = rank < CAP

    slot_of_token = jnp.where(kept, r_all * CAP + rank, E * CAP)
    tfs = (
        jnp.zeros((E * CAP + 1,), jnp.int32)
        .at[slot_of_token]
        .set(jnp.arange(N_DEV * N_PER, dtype=jnp.int32), mode="drop")[: E * CAP]
    )
    my_tfs = lax.dynamic_slice(tfs, (my * SLOTS,), (SLOTS,))
    xg = jnp.take(x_all, my_tfs, axis=0)

    y_all = _expert_mm_ag(xg, expert_W)

    r_mine = lax.dynamic_slice(r_all, (my * N_PER,), (N_PER,))
    rank_mine = lax.dynamic_slice(rank, (my * N_PER,), (N_PER,))
    kept_mine = rank_mine < CAP
    g = jnp.where(kept_mine, r_mine * CAP + rank_mine, 0)
    out = jnp.where(
        kept_mine[:, None], jnp.take(y_all, g, axis=0).astype(jnp.float32), 0.0
    )
    return out


# baseline (device time: 366232 ns/iter reference)
import jax
import jax.numpy as jnp
from jax import lax
from jax.experimental import pallas as pl
from jax.experimental.pallas import tpu as pltpu

N_DEV = 4
N_PER = 2048
D = 1024
E = 32
E_LOC = 8
CAP = 204
CAPP = 208
SLOTS = E_LOC * CAPP
R_ROWS = N_PER // 128


def _ring_ag(x, r2):

    def body(x_ref, r_ref, xall_ref, rall_ref, sx, rx, sr, rr):
        my = lax.axis_index("i")
        left = lax.rem(my + N_DEV - 1, N_DEV)
        right = lax.rem(my + 1, N_DEV)

        barrier_sem = pltpu.get_barrier_semaphore()
        for nbr in (left, right):
            pl.semaphore_signal(
                barrier_sem, inc=1, device_id=(nbr,),
                device_id_type=pl.DeviceIdType.MESH,
            )
        pl.semaphore_wait(barrier_sem, 2)

        xall_ref[pl.ds(my * N_PER, N_PER), :] = x_ref[:, :].astype(jnp.bfloat16)
        rall_ref[pl.ds(my * R_ROWS, R_ROWS), :] = r_ref[:, :]

        for h in range(N_DEV - 1):
            sb = lax.rem(my + N_DEV - h, N_DEV)
            rdma_x = pltpu.make_async_remote_copy(
                src_ref=xall_ref.at[pl.ds(sb * N_PER, N_PER), :],
                dst_ref=xall_ref.at[pl.ds(sb * N_PER, N_PER), :],
                send_sem=sx.at[h],
                recv_sem=rx.at[h],
                device_id=(right,),
                device_id_type=pl.DeviceIdType.MESH,
            )
            rdma_r = pltpu.make_async_remote_copy(
                src_ref=rall_ref.at[pl.ds(sb * R_ROWS, R_ROWS), :],
                dst_ref=rall_ref.at[pl.ds(sb * R_ROWS, R_ROWS), :],
                send_sem=sr.at[h],
                recv_sem=rr.at[h],
                device_id=(right,),
                device_id_type=pl.DeviceIdType.MESH,
            )
            rdma_x.start()
            rdma_r.start()
            rdma_x.wait()
            rdma_r.wait()

    return pl.pallas_call(
        body,
        out_shape=(
            jax.ShapeDtypeStruct((N_DEV * N_PER, D), jnp.bfloat16),
            jax.ShapeDtypeStruct((N_DEV * R_ROWS, 128), jnp.int32),
        ),
        in_specs=[
            pl.BlockSpec(memory_space=pltpu.VMEM),
            pl.BlockSpec(memory_space=pltpu.VMEM),
        ],
        out_specs=(
            pl.BlockSpec(memory_space=pltpu.VMEM),
            pl.BlockSpec(memory_space=pltpu.VMEM),
        ),
        scratch_shapes=[
            pltpu.SemaphoreType.DMA((N_DEV - 1,)),
            pltpu.SemaphoreType.DMA((N_DEV - 1,)),
            pltpu.SemaphoreType.DMA((N_DEV - 1,)),
            pltpu.SemaphoreType.DMA((N_DEV - 1,)),
        ],
        compiler_params=pltpu.CompilerParams(collective_id=0),
    )(x, r2)


def _expert_mm_ag(xg, expert_W):

    def body(xg_ref, ew_ref, yall_ref, wbuf, csem, ss, rs):
        my = lax.axis_index("i")
        left = lax.rem(my + N_DEV - 1, N_DEV)
        right = lax.rem(my + 1, N_DEV)

        barrier_sem = pltpu.get_barrier_semaphore()
        for nbr in (left, right):
            pl.semaphore_signal(
                barrier_sem, inc=1, device_id=(nbr,),
                device_id_type=pl.DeviceIdType.MESH,
            )
        pl.semaphore_wait(barrier_sem, 2)

        for le in range(E_LOC):
            cp = pltpu.make_async_copy(ew_ref.at[le], wbuf, csem)
            cp.start()
            cp.wait()
            w16 = wbuf[:, :].astype(jnp.bfloat16)
            y = jnp.dot(
                xg_ref[pl.ds(le * CAPP, CAPP), :], w16,
                preferred_element_type=jnp.float32,
            )
            yall_ref[pl.ds(my * SLOTS + le * CAPP, CAPP), :] = y.astype(jnp.bfloat16)

        for h in range(N_DEV - 1):
            sb = lax.rem(my + N_DEV - h, N_DEV)
            rdma = pltpu.make_async_remote_copy(
                src_ref=yall_ref.at[pl.ds(sb * SLOTS, SLOTS), :],
                dst_ref=yall_ref.at[pl.ds(sb * SLOTS, SLOTS), :],
                send_sem=ss.at[h],
                recv_sem=rs.at[h],
                device_id=(right,),
                device_id_type=pl.DeviceIdType.MESH,
            )
            rdma.start()
            rdma.wait()

    return pl.pallas_call(
        body,
        out_shape=jax.ShapeDtypeStruct((N_DEV * SLOTS, D), jnp.bfloat16),
        in_specs=[
            pl.BlockSpec(memory_space=pltpu.VMEM),
            pl.BlockSpec(memory_space=pl.ANY),
        ],
        out_specs=pl.BlockSpec(memory_space=pltpu.VMEM),
        scratch_shapes=[
            pltpu.VMEM((D, D), jnp.float32),
            pltpu.SemaphoreType.DMA,
            pltpu.SemaphoreType.DMA((N_DEV - 1,)),
            pltpu.SemaphoreType.DMA((N_DEV - 1,)),
        ],
        compiler_params=pltpu.CompilerParams(collective_id=1),
    )(xg, expert_W)


def kernel(x, router_W, route_idx, expert_W):
    del router_W
    my = lax.axis_index("i")

    x_all, rall = _ring_ag(x, route_idx.reshape(R_ROWS, 128))
    r_all = rall.reshape(N_DEV * N_PER)

    oh = (r_all[:, None] == jnp.arange(E, dtype=jnp.int32)[None, :]).astype(jnp.int32)
    rank = jnp.take_along_axis(jnp.cumsum(oh, axis=0), r_all[:, None], axis=1)[:, 0] - 1
    kept = rank < CAP

    slot_of_token = jnp.where(kept, r_all * CAPP + rank, E * CAPP)
    tfs = (
        jnp.zeros((E * CAPP + 1,), jnp.int32)
        .at[slot_of_token]
        .set(jnp.arange(N_DEV * N_PER, dtype=jnp.int32), mode="drop")[: E * CAPP]
    )
    my_tfs = lax.dynamic_slice(tfs, (my * SLOTS,), (SLOTS,))
    xg = jnp.take(x_all, my_tfs, axis=0)

    y_all = _expert_mm_ag(xg, expert_W)

    r_mine = lax.dynamic_slice(r_all, (my * N_PER,), (N_PER,))
    rank_mine = lax.dynamic_slice(rank, (my * N_PER,), (N_PER,))
    kept_mine = rank_mine < CAP
    g = jnp.where(kept_mine, r_mine * CAPP + rank_mine, 0)
    out = jnp.where(
        kept_mine[:, None], jnp.take(y_all, g, axis=0).astype(jnp.float32), 0.0
    )
    return out


# device time: 361849 ns/iter; 1.0121x vs baseline; 1.0121x over previous
import jax
import jax.numpy as jnp
from jax import lax
from jax.experimental import pallas as pl
from jax.experimental.pallas import tpu as pltpu

N_DEV = 4
N_PER = 2048
D = 1024
E = 32
E_LOC = 8
CAP = 204
CAPP = 208
SLOTS = E_LOC * CAPP
R_ROWS = N_PER // 128


def _ring_ag(x, r2):

    def body(x_ref, r_ref, xall_ref, rall_ref, sx, rx, sr, rr):
        my = lax.axis_index("i")
        left = lax.rem(my + N_DEV - 1, N_DEV)
        right = lax.rem(my + 1, N_DEV)

        barrier_sem = pltpu.get_barrier_semaphore()
        for nbr in (left, right):
            pl.semaphore_signal(
                barrier_sem, inc=1, device_id=(nbr,),
                device_id_type=pl.DeviceIdType.MESH,
            )
        pl.semaphore_wait(barrier_sem, 2)

        xall_ref[pl.ds(my * N_PER, N_PER), :] = x_ref[:, :].astype(jnp.bfloat16)
        rall_ref[pl.ds(my * R_ROWS, R_ROWS), :] = r_ref[:, :]

        for h in range(N_DEV - 1):
            sb = lax.rem(my + N_DEV - h, N_DEV)
            rdma_x = pltpu.make_async_remote_copy(
                src_ref=xall_ref.at[pl.ds(sb * N_PER, N_PER), :],
                dst_ref=xall_ref.at[pl.ds(sb * N_PER, N_PER), :],
                send_sem=sx.at[h],
                recv_sem=rx.at[h],
                device_id=(right,),
                device_id_type=pl.DeviceIdType.MESH,
            )
            rdma_r = pltpu.make_async_remote_copy(
                src_ref=rall_ref.at[pl.ds(sb * R_ROWS, R_ROWS), :],
                dst_ref=rall_ref.at[pl.ds(sb * R_ROWS, R_ROWS), :],
                send_sem=sr.at[h],
                recv_sem=rr.at[h],
                device_id=(right,),
                device_id_type=pl.DeviceIdType.MESH,
            )
            rdma_x.start()
            rdma_r.start()
            rdma_x.wait()
            rdma_r.wait()

    return pl.pallas_call(
        body,
        out_shape=(
            jax.ShapeDtypeStruct((N_DEV * N_PER, D), jnp.bfloat16),
            jax.ShapeDtypeStruct((N_DEV * R_ROWS, 128), jnp.int32),
        ),
        in_specs=[
            pl.BlockSpec(memory_space=pltpu.VMEM),
            pl.BlockSpec(memory_space=pltpu.VMEM),
        ],
        out_specs=(
            pl.BlockSpec(memory_space=pltpu.VMEM),
            pl.BlockSpec(memory_space=pltpu.VMEM),
        ),
        scratch_shapes=[
            pltpu.SemaphoreType.DMA((N_DEV - 1,)),
            pltpu.SemaphoreType.DMA((N_DEV - 1,)),
            pltpu.SemaphoreType.DMA((N_DEV - 1,)),
            pltpu.SemaphoreType.DMA((N_DEV - 1,)),
        ],
        compiler_params=pltpu.CompilerParams(collective_id=0),
    )(x, r2)


def _expert_mm_ag(x_all, rr2, rk2, expert_W):

    def body(x_ref, rr_ref, rk_ref, ew_ref, yall_ref, wbuf, csem, ss, rs):
        my = lax.axis_index("i")
        left = lax.rem(my + N_DEV - 1, N_DEV)
        right = lax.rem(my + 1, N_DEV)

        barrier_sem = pltpu.get_barrier_semaphore()
        for nbr in (left, right):
            pl.semaphore_signal(
                barrier_sem, inc=1, device_id=(nbr,),
                device_id_type=pl.DeviceIdType.MESH,
            )
        pl.semaphore_wait(barrier_sem, 2)

        def expert_step(le, _):
            cp = pltpu.make_async_copy(ew_ref.at[le], wbuf, csem)
            cp.start()
            e = my * E_LOC + le
            rr = rr_ref[0:1, :]
            rk = rk_ref[0:1, :]
            c_iota = lax.broadcasted_iota(jnp.int32, (CAPP, N_DEV * N_PER), 0)
            mine = (rr == e) & (rk < CAP)
            onehot = ((c_iota == rk) & mine).astype(jnp.bfloat16)
            xd = jnp.dot(
                onehot, x_ref[:, :], preferred_element_type=jnp.float32
            ).astype(jnp.bfloat16)
            cp.wait()
            w16 = wbuf[:, :].astype(jnp.bfloat16)
            y = jnp.dot(xd, w16, preferred_element_type=jnp.float32)
            yall_ref[pl.ds(my * SLOTS + le * CAPP, CAPP), :] = y.astype(jnp.bfloat16)
            return 0

        lax.fori_loop(0, E_LOC, expert_step, 0)

        for h in range(N_DEV - 1):
            sb = lax.rem(my + N_DEV - h, N_DEV)
            rdma = pltpu.make_async_remote_copy(
                src_ref=yall_ref.at[pl.ds(sb * SLOTS, SLOTS), :],
                dst_ref=yall_ref.at[pl.ds(sb * SLOTS, SLOTS), :],
                send_sem=ss.at[h],
                recv_sem=rs.at[h],
                device_id=(right,),
                device_id_type=pl.DeviceIdType.MESH,
            )
            rdma.start()
            rdma.wait()

    return pl.pallas_call(
        body,
        out_shape=jax.ShapeDtypeStruct((N_DEV * SLOTS, D), jnp.bfloat16),
        in_specs=[
            pl.BlockSpec(memory_space=pltpu.VMEM),
            pl.BlockSpec(memory_space=pltpu.VMEM),
            pl.BlockSpec(memory_space=pltpu.VMEM),
            pl.BlockSpec(memory_space=pl.ANY),
        ],
        out_specs=pl.BlockSpec(memory_space=pltpu.VMEM),
        scratch_shapes=[
            pltpu.VMEM((D, D), jnp.float32),
            pltpu.SemaphoreType.DMA,
            pltpu.SemaphoreType.DMA((N_DEV - 1,)),
            pltpu.SemaphoreType.DMA((N_DEV - 1,)),
        ],
        compiler_params=pltpu.CompilerParams(collective_id=1),
    )(x_all, rr2, rk2, expert_W)


def kernel(x, router_W, route_idx, expert_W):
    del router_W
    my = lax.axis_index("i")

    x_all, rall = _ring_ag(x, route_idx.reshape(R_ROWS, 128))
    r_all = rall.reshape(N_DEV * N_PER)

    oh = (r_all[:, None] == jnp.arange(E, dtype=jnp.int32)[None, :]).astype(jnp.int32)
    rank = jnp.take_along_axis(jnp.cumsum(oh, axis=0), r_all[:, None], axis=1)[:, 0] - 1

    rr2 = r_all.reshape(1, N_DEV * N_PER)
    rk2 = rank.reshape(1, N_DEV * N_PER)
    y_all = _expert_mm_ag(x_all, rr2, rk2, expert_W)

    r_mine = lax.dynamic_slice(r_all, (my * N_PER,), (N_PER,))
    rank_mine = lax.dynamic_slice(rank, (my * N_PER,), (N_PER,))
    kept_mine = rank_mine < CAP
    g = jnp.where(kept_mine, r_mine * CAPP + rank_mine, 0)
    out = jnp.where(
        kept_mine[:, None], jnp.take(y_all, g, axis=0).astype(jnp.float32), 0.0
    )
    return out
